# baseline (speedup 1.0000x reference)
import functools

import jax
import jax.numpy as jnp
from jax import lax
from jax.experimental import pallas as pl
from jax.experimental.pallas import tpu as pltpu
from jax.experimental.pallas import tpu_sc as plsc

_PREC = lax.Precision.DEFAULT

N = 10000
E = 320000
B = 256
DH = 128
L = 3

NT = 16
CH = 128
NBK = 2 * NT
BROWS = 320
NPAD = NBK * BROWS
RPT = NPAD // NT
NPAD2 = N + 16
EPADS = E + NBK * CH
NCHT = EPADS // CH
ZR = 16


def _seg_sum_pairs(x2, src2, dst2, meta2):
    mesh = plsc.VectorSubcoreMesh(core_axis_name="c", subcore_axis_name="s")

    @functools.partial(
        pl.kernel,
        out_type=jax.ShapeDtypeStruct((2, NPAD, DH), jnp.float32),
        mesh=mesh,
        scratch_types=[
            pltpu.VMEM((CH,), jnp.int32),
            pltpu.VMEM((CH,), jnp.int32),
            pltpu.VMEM((CH, DH), jnp.float32),
            pltpu.VMEM((ZR, DH), jnp.float32),
            pltpu.VMEM((16,), jnp.int32),
            pltpu.VMEM_SHARED((NPAD, DH), jnp.float32),
            pltpu.SemaphoreType.DMA,
        ],
    )
    def k(x_hbm, src_hbm, dst_hbm, meta_hbm, out_hbm,
          src_v, dst_v, rows_v, zbuf, meta_v, acc, sem):
        c = lax.axis_index("c")
        s = lax.axis_index("s")

        for r in range(ZR):
            for g in range(DH // 16):
                zbuf[r, pl.ds(g * 16, 16)] = jnp.zeros((16,), jnp.float32)
        row0 = s * RPT

        def zbody(i, _):
            pltpu.sync_copy(zbuf, acc.at[pl.ds(row0 + i * ZR, ZR)])
            return 0

        lax.fori_loop(0, RPT // ZR, zbody, 0)
        plsc.subcore_barrier()

        pltpu.sync_copy(meta_hbm.at[c, s], meta_v)
        mv = meta_v[...]

        def body(j, pch):
            ch = pch + j
            pltpu.sync_copy(src_hbm.at[c, ch], src_v)
            pltpu.sync_copy(dst_hbm.at[c, ch], dst_v)
            pltpu.async_copy(x_hbm.at[c].at[src_v], rows_v, sem).wait()
            pltpu.sync_copy(rows_v, acc.at[dst_v], add=True)
            return pch

        for b in range(2):
            lax.fori_loop(0, mv[2 * b + 1], body, mv[2 * b])

        plsc.subcore_barrier()
        pltpu.sync_copy(acc.at[pl.ds(row0, RPT)],
                        out_hbm.at[c, pl.ds(row0, RPT)])

    return k(x2, src2, dst2, meta2)


def _prep_edges(so_ei, sv_ei):
    i32 = jnp.int32
    src2r = jnp.stack([so_ei[0], sv_ei[0]])
    dst2r = jnp.stack([so_ei[1], sv_ei[1]])
    order = jnp.argsort(dst2r, axis=1, stable=True)
    src_s = jnp.take_along_axis(src2r, order, axis=1)
    dst_s = jnp.take_along_axis(dst2r, order, axis=1)
    bounds = (jnp.arange(NBK + 1, dtype=i32) * BROWS)
    starts = jax.vmap(
        lambda a: jnp.searchsorted(a, bounds, side="left"))(dst_s)
    starts = starts.astype(i32)
    counts = starts[:, 1:] - starts[:, :-1]
    nch = (counts + (CH - 1)) // CH
    pch = jnp.concatenate(
        [jnp.zeros((2, 1), i32), jnp.cumsum(nch, axis=1, dtype=i32)], axis=1)
    bucket = dst_s // BROWS
    pchb = jnp.take_along_axis(pch, bucket, axis=1)
    stb = jnp.take_along_axis(starts, bucket, axis=1)
    k = jnp.arange(E, dtype=i32)[None, :]
    newpos = pchb * CH + (k - stb)
    flat = (newpos + jnp.array([[0], [EPADS]], i32)).reshape(-1)
    kk = jnp.arange(EPADS, dtype=i32)
    src_fill = jnp.tile(N + (kk % (NPAD2 - N)), 2)
    dst_fill = jnp.tile(kk % NPAD, 2)
    src_p = src_fill.at[flat].set(
        src_s.reshape(-1), unique_indices=True, indices_are_sorted=True)
    dst_p = dst_fill.at[flat].set(
        dst_s.reshape(-1), unique_indices=True, indices_are_sorted=True)
    src2 = src_p.reshape(2, NCHT, CH)
    dst2 = dst_p.reshape(2, NCHT, CH)
    pch_t = pch[:, :NBK].reshape(2, NT, 2)
    nch_t = nch.reshape(2, NT, 2)
    meta2 = jnp.stack(
        [pch_t[..., 0], nch_t[..., 0], pch_t[..., 1], nch_t[..., 1]],
        axis=-1)
    meta2 = jnp.concatenate(
        [meta2, jnp.zeros((2, NT, 12), i32)], axis=-1)
    return src2, dst2, meta2


def _bn(h, g, b):
    m = jnp.mean(h, axis=0, keepdims=True)
    d = h - m
    v = jnp.mean(d * d, axis=0, keepdims=True)
    return g * d / jnp.sqrt(v + 1e-5) + b


def _lin0_body(x_ref, wt_ref, b_ref, o_ref):
    o_ref[0, :N] = (
        jnp.dot(x_ref[0], wt_ref[0], preferred_element_type=jnp.float32,
                precision=_PREC)
        + b_ref[0]
    )
    o_ref[0, N:] = jnp.zeros((NPAD2 - N, DH), jnp.float32)


def _gin_dense(x, agg, eps, w1t, b1, g1, t1, w2t, b2, g2, t2):
    h = (1.0 + eps) * x + agg
    h = jnp.dot(h, w1t, preferred_element_type=jnp.float32,
                precision=_PREC) + b1
    h = jnp.maximum(_bn(h, g1, t1), 0.0)
    h = jnp.dot(h, w2t, preferred_element_type=jnp.float32,
                precision=_PREC) + b2
    return jnp.maximum(_bn(h, g2, t2), 0.0)


def _gin_body(x_ref, agg_ref, eps_ref, w1t_ref, b1_ref, g1_ref, t1_ref,
              w2t_ref, b2_ref, g2_ref, t2_ref, o_ref):
    o_ref[0, :N] = _gin_dense(
        x_ref[0, :N], agg_ref[0, :N, :], eps_ref[0], w1t_ref[0], b1_ref[0],
        g1_ref[0], t1_ref[0], w2t_ref[0], b2_ref[0], g2_ref[0], t2_ref[0])
    o_ref[0, N:] = jnp.zeros((NPAD2 - N, DH), jnp.float32)


def _final_body(x_ref, agg_ref, batch_ref, tm_ref, eps_ref, w1t_ref, b1_ref,
                g1_ref, t1_ref, w2t_ref, b2_ref, g2_ref, t2_ref,
                wa_ref, wb_ref, wc_ref, mb0_ref, mg0_ref, mt0_ref,
                mw1t_ref, mb1_ref, mg1_ref, mt1_ref, mw2t_ref, mb2_ref,
                o_ref):
    pools = []
    for t in range(2):
        h = _gin_dense(
            x_ref[t, :N], agg_ref[t, :N, :], eps_ref[t], w1t_ref[t],
            b1_ref[t], g1_ref[t], t1_ref[t], w2t_ref[t], b2_ref[t],
            g2_ref[t], t2_ref[t])
        bt = batch_ref[t]
        oh = (bt == lax.broadcasted_iota(jnp.int32, (N, B), 1)).astype(
            jnp.float32)
        s = lax.dot_general(oh, h, (((0,), (0,)), ((), ())),
                            preferred_element_type=jnp.float32,
                            precision=lax.Precision.HIGHEST)
        cnt = lax.dot_general(oh, jnp.ones((N, 1), jnp.float32),
                              (((0,), (0,)), ((), ())),
                              preferred_element_type=jnp.float32,
                              precision=lax.Precision.HIGHEST)
        pools.append(s / jnp.maximum(cnt, 1.0))
    a = (
        jnp.dot(pools[0], wa_ref[...], preferred_element_type=jnp.float32,
                precision=_PREC)
        + jnp.dot(pools[1], wb_ref[...], preferred_element_type=jnp.float32,
                  precision=_PREC)
        + jnp.dot(tm_ref[...], wc_ref[...], preferred_element_type=jnp.float32,
                  precision=_PREC)
        + mb0_ref[...]
    )
    a = _bn(a, mg0_ref[...], mt0_ref[...])
    a = jnp.where(a >= 0.0, a, 0.01 * a)
    a = jnp.dot(a, mw1t_ref[...], preferred_element_type=jnp.float32,
                precision=_PREC) + mb1_ref[...]
    a = _bn(a, mg1_ref[...], mt1_ref[...])
    a = jnp.where(a >= 0.0, a, 0.01 * a)
    o_ref[...] = (
        jnp.dot(a, mw2t_ref[...], preferred_element_type=jnp.float32,
                precision=_PREC)
        + mb2_ref[...]
    )


def _tower_spec(shape):
    nd = len(shape)
    return pl.BlockSpec((1,) + shape, lambda t, _n=nd: (t,) + (0,) * _n)


def kernel(solute_x, solute_edge_index, solute_batch, solvent_x,
           solvent_edge_index, solvent_batch, tm,
           so_lin0_W, so_lin0_b, so_gin_W1, so_gin_b1, so_gin_W2, so_gin_b2,
           so_gin_g1, so_gin_bt1, so_gin_g2, so_gin_bt2, so_eps,
           sv_lin0_W, sv_lin0_b, sv_gin_W1, sv_gin_b1, sv_gin_W2, sv_gin_b2,
           sv_gin_g1, sv_gin_bt1, sv_gin_g2, sv_gin_bt2, sv_eps,
           mlp_W0, mlp_b0, mlp_g0, mlp_bt0, mlp_W1, mlp_b1, mlp_g1, mlp_bt1,
           mlp_W2, mlp_b2):
    f32 = jnp.float32

    x2in = jnp.stack([solute_x, solvent_x])
    src2, dst2, meta2 = _prep_edges(solute_edge_index, solvent_edge_index)
    batch2 = jnp.stack([solute_batch.reshape(N, 1),
                        solvent_batch.reshape(N, 1)])

    w0t = jnp.stack([so_lin0_W.T, sv_lin0_W.T])
    b0 = jnp.stack([so_lin0_b.reshape(1, DH), sv_lin0_b.reshape(1, DH)])

    def lw(i):
        return dict(
            eps=jnp.stack([so_eps[i].reshape(1, 1), sv_eps[i].reshape(1, 1)]),
            w1t=jnp.stack([so_gin_W1[i].T, sv_gin_W1[i].T]),
            b1=jnp.stack([so_gin_b1[i].reshape(1, DH), sv_gin_b1[i].reshape(1, DH)]),
            g1=jnp.stack([so_gin_g1[i].reshape(1, DH), sv_gin_g1[i].reshape(1, DH)]),
            t1=jnp.stack([so_gin_bt1[i].reshape(1, DH), sv_gin_bt1[i].reshape(1, DH)]),
            w2t=jnp.stack([so_gin_W2[i].T, sv_gin_W2[i].T]),
            b2=jnp.stack([so_gin_b2[i].reshape(1, DH), sv_gin_b2[i].reshape(1, DH)]),
            g2=jnp.stack([so_gin_g2[i].reshape(1, DH), sv_gin_g2[i].reshape(1, DH)]),
            t2=jnp.stack([so_gin_bt2[i].reshape(1, DH), sv_gin_bt2[i].reshape(1, DH)]),
        )

    mw0t = mlp_W0.T
    wa, wb, wc = mw0t[:DH], mw0t[DH:2 * DH], mw0t[2 * DH:]
    mw1t = mlp_W1.T
    mw2t = mlp_W2.T

    x2 = pl.pallas_call(
        _lin0_body,
        grid=(2,),
        in_specs=[_tower_spec((N, DH)), _tower_spec((DH, DH)),
                  _tower_spec((1, DH))],
        out_specs=_tower_spec((NPAD2, DH)),
        out_shape=jax.ShapeDtypeStruct((2, NPAD2, DH), f32),
    )(x2in, w0t, b0)

    for i in range(L - 1):
        agg = _seg_sum_pairs(x2, src2, dst2, meta2)
        p = lw(i)
        x2 = pl.pallas_call(
            _gin_body,
            grid=(2,),
            in_specs=[_tower_spec((NPAD2, DH)), _tower_spec((NPAD, DH)),
                      _tower_spec((1, 1)), _tower_spec((DH, DH)),
                      _tower_spec((1, DH)), _tower_spec((1, DH)),
                      _tower_spec((1, DH)), _tower_spec((DH, DH)),
                      _tower_spec((1, DH)), _tower_spec((1, DH)),
                      _tower_spec((1, DH))],
            out_specs=_tower_spec((NPAD2, DH)),
            out_shape=jax.ShapeDtypeStruct((2, NPAD2, DH), f32),
        )(x2, agg, p["eps"], p["w1t"], p["b1"], p["g1"], p["t1"],
          p["w2t"], p["b2"], p["g2"], p["t2"])

    agg = _seg_sum_pairs(x2, src2, dst2, meta2)
    p = lw(L - 1)
    out = pl.pallas_call(
        _final_body,
        out_shape=jax.ShapeDtypeStruct((B, 1), f32),
    )(x2, agg, batch2, tm, p["eps"], p["w1t"], p["b1"], p["g1"], p["t1"],
      p["w2t"], p["b2"], p["g2"], p["t2"],
      wa, wb, wc, mlp_b0.reshape(1, 105), mlp_g0.reshape(1, 105),
      mlp_bt0.reshape(1, 105), mw1t, mlp_b1.reshape(1, 74),
      mlp_g1.reshape(1, 74), mlp_bt1.reshape(1, 74), mw2t,
      mlp_b2.reshape(1, 1))
    return out
